# baseline (speedup 1.0000x reference)
import functools

import jax
import jax.numpy as jnp
from jax import lax
from jax.experimental import pallas as pl
from jax.experimental.pallas import tpu as pltpu
from jax.experimental.pallas import tpu_sc as plsc

N = 10000
E = 320000
D = 128
DH = D // 2

NC = 2
NS = 16
NW = NC * NS
EPW = E // NW
EPT = E // NS
CH = 100
NSEG = 5
SEG = 40
KB = 5
RPT = 624
TAIL0 = NS * RPT
TAIL = N - TAIL0
CHX = 80
NXCH = (N + CHX - 1) // CHX

_mesh = plsc.VectorSubcoreMesh(core_axis_name="c", subcore_axis_name="s")


def _a_body(x_hbm, ni_hbm, e1d_hbm, e2d_hbm,
            xn_hbm, dp1_hbm, dp2_hbm,
            idx_v, rows_v, cnt_v, deg_v, sem):
    c = lax.axis_index("c")
    s = lax.axis_index("s")
    wid = s * NC + c

    for k in range((NXCH + NW - 1) // NW):
        jj = wid + NW * k

        @pl.when(jj < NXCH)
        def _():
            off = pl.multiple_of(jj * CHX, 8)
            pltpu.sync_copy(ni_hbm.at[pl.ds(off, CHX)], idx_v)
            pltpu.async_copy(x_hbm.at[idx_v], rows_v, sem).wait()
            pltpu.sync_copy(rows_v, xn_hbm.at[pl.ds(off, CHX)])

    ones16 = jnp.full((16,), 1.0, jnp.float32)
    zeros16 = jnp.zeros((16,), jnp.float32)
    for ed_hbm, dp_hbm in ((e1d_hbm, dp1_hbm), (e2d_hbm, dp2_hbm)):
        def zero_body(i, carry):
            deg_v[pl.ds(i * 16, 16)] = zeros16
            return carry
        lax.fori_loop(0, N // 16, zero_body, 0)
        pltpu.sync_copy(ed_hbm.at[wid], cnt_v)

        def cnt_body(i, carry):
            idx = cnt_v[i, :]
            plsc.addupdate_scatter(deg_v, [idx], ones16)
            return carry
        lax.fori_loop(0, EPW // 16, cnt_body, 0)
        pltpu.sync_copy(deg_v, dp_hbm.at[pl.ds(pl.multiple_of(wid * N, 8), N)])


_a_kernel = functools.partial(
    pl.kernel,
    out_type=(
        jax.ShapeDtypeStruct((N, D), jnp.float32),
        jax.ShapeDtypeStruct((NW * N,), jnp.float32),
        jax.ShapeDtypeStruct((NW * N,), jnp.float32),
    ),
    mesh=_mesh,
    scratch_types=[
        pltpu.VMEM((CHX,), jnp.int32),
        pltpu.VMEM((CHX, D), jnp.float32),
        pltpu.VMEM((EPW // 16, 16), jnp.int32),
        pltpu.VMEM((N,), jnp.float32),
        pltpu.SemaphoreType.DMA,
    ],
    compiler_params=pltpu.CompilerParams(needs_layout_passes=False),
)(_a_body)


RB = 400
NB = N // RB


def _b_body(x_b, xn_b, w1_b, w2_b, dp1_b, dp2_b,
            g1_b, gn1_b, g2_b, gn2_b, dv1_b, dv2_b):
    deg1 = jnp.sum(dp1_b[...], axis=1, keepdims=True) + 1.0
    deg2 = jnp.sum(dp2_b[...], axis=1, keepdims=True) + 1.0
    dv1 = lax.rsqrt(deg1)
    dv2 = lax.rsqrt(deg2)
    dv1_b[...] = dv1
    dv2_b[...] = dv2
    x = x_b[...]
    xn = xn_b[...]
    w1 = w1_b[...]
    w2 = w2_b[...]
    for out_b, rows, w, dv in ((g1_b, x, w1, dv1), (gn1_b, xn, w1, dv1),
                               (g2_b, x, w2, dv2), (gn2_b, xn, w2, dv2)):
        g = jnp.dot(rows, w, preferred_element_type=jnp.float32) * dv
        out_b[0, :, :] = g[:, :DH]
        out_b[1, :, :] = g[:, DH:]


def _b_call(x, xn, W1, W2, dp1, dp2):
    row_spec = pl.BlockSpec((RB, D), lambda i: (i, 0))
    w_spec = pl.BlockSpec((D, D), lambda i: (0, 0))
    dp_spec = pl.BlockSpec((RB, NW), lambda i: (i, 0))
    g_spec = pl.BlockSpec((2, RB, DH), lambda i: (0, i, 0))
    dv_spec = pl.BlockSpec((RB, 1), lambda i: (i, 0))
    g_shape = jax.ShapeDtypeStruct((2, N, DH), jnp.float32)
    return pl.pallas_call(
        _b_body,
        grid=(NB,),
        in_specs=[row_spec, row_spec, w_spec, w_spec, dp_spec, dp_spec],
        out_specs=[g_spec, g_spec, g_spec, g_spec, dv_spec, dv_spec],
        out_shape=[
            g_shape, g_shape, g_shape, g_shape,
            jax.ShapeDtypeStruct((N, 1), jnp.float32),
            jax.ShapeDtypeStruct((N, 1), jnp.float32),
        ],
    )(x, xn, W1, W2, dp1, dp2)


def _c_body(g1, gn1, g2, gn2, zeros_hbm, s1x, d1x, s2x, d2x,
            raw, src_v, dst_v, rows, acc_sp, gsems, ssems):
    c = lax.axis_index("c")
    s = lax.axis_index("s")
    row0 = pl.multiple_of(s * RPT, 8)
    branches = ((g1, s1x, d1x), (gn1, s1x, d1x), (g2, s2x, d2x),
                (gn2, s2x, d2x))

    def _blk_copy(copy_fn):
        copy_fn(row0, RPT)

        @pl.when(s == 0)
        def _():
            copy_fn(TAIL0, TAIL)

    _blk_copy(lambda r, n: pltpu.sync_copy(
        zeros_hbm.at[pl.ds(r, n)], acc_sp.at[pl.ds(r, n)]))
    plsc.subcore_barrier()

    for b, (tbl, sx, dx) in enumerate(branches):
        for seg in range(NSEG):
            pltpu.sync_copy(sx.at[c, s, seg], src_v)
            pltpu.sync_copy(dx.at[s, seg], dst_v)
            for k in range(KB - 1):
                pltpu.async_copy(tbl.at[src_v.at[k]], rows[k], gsems[k])

            def block(ib, carry):
                jb = ib * KB
                for k in range(KB):
                    j = jb + k
                    q = (k - 1) % KB
                    pltpu.make_async_copy(
                        tbl.at[src_v.at[j]], rows[k], gsems[k]).wait()
                    pltpu.async_copy(
                        rows[k], acc_sp.at[dst_v.at[j]], ssems[k], add=True)

                    @pl.when(j > 0)
                    def _():
                        pltpu.make_async_copy(
                            rows[q], acc_sp.at[dst_v.at[j]], ssems[q]).wait()

                    @pl.when(j + KB - 1 < SEG)
                    def _():
                        pltpu.async_copy(
                            tbl.at[src_v.at[j + KB - 1]], rows[q], gsems[q])
                return carry
            lax.fori_loop(0, SEG // KB, block, 0)
            pltpu.make_async_copy(
                rows[(SEG - 1) % KB], acc_sp.at[dst_v.at[SEG - 1]],
                ssems[(SEG - 1) % KB]).wait()
        plsc.subcore_barrier()

        _blk_copy(lambda r, n: pltpu.sync_copy(
            acc_sp.at[pl.ds(r, n)], raw.at[b, c, pl.ds(r, n)]))
        plsc.subcore_barrier()


_c_kernel = functools.partial(
    pl.kernel,
    out_type=jax.ShapeDtypeStruct((4, NC, N, DH), jnp.float32),
    mesh=_mesh,
    scratch_types=[
        pltpu.VMEM((SEG, CH), jnp.int32),
        pltpu.VMEM((SEG, CH), jnp.int32),
        [pltpu.VMEM((CH, DH), jnp.float32) for _ in range(KB)],
        pltpu.VMEM_SHARED((N, DH), jnp.float32),
        [pltpu.SemaphoreType.DMA for _ in range(KB)],
        [pltpu.SemaphoreType.DMA for _ in range(KB)],
    ],
    compiler_params=pltpu.CompilerParams(use_tc_tiling_on_sc=False),
)(_c_body)


def _d_body(r00_b, r01_b, r10_b, r11_b, r20_b, r21_b, r30_b, r31_b,
            g1l_b, g1h_b, gn1l_b, gn1h_b, g2l_b, g2h_b, gn2l_b, gn2h_b,
            dv1_b, dv2_b, b1_b, b2_b,
            p1_b, n1_b, p2_b, n2_b, s1_b, s2_b):
    i = pl.program_id(0)
    dv1 = dv1_b[...]
    dv2 = dv2_b[...]
    b1 = b1_b[...]
    b2 = b2_b[...]

    def cat(lo, hi):
        return jnp.concatenate([lo, hi], axis=1)

    r0 = cat(r00_b[0, 0], r01_b[0, 0])
    r1 = cat(r10_b[0, 0], r11_b[0, 0])
    r2 = cat(r20_b[0, 0], r21_b[0, 0])
    r3 = cat(r30_b[0, 0], r31_b[0, 0])
    g1 = cat(g1l_b[0], g1h_b[0])
    gn1 = cat(gn1l_b[0], gn1h_b[0])
    g2 = cat(g2l_b[0], g2h_b[0])
    gn2 = cat(gn2l_b[0], gn2h_b[0])
    p1 = jnp.maximum(dv1 * (r0 + g1) + b1, 0.0)
    n1 = jnp.maximum(dv1 * (r1 - r0 + gn1) + b1, 0.0)
    p2 = jnp.maximum(dv2 * (r2 - r1 + g2) + b2, 0.0)
    n2 = jnp.maximum(dv2 * (r3 - r2 + gn2) + b2, 0.0)
    p1_b[...] = p1
    n1_b[...] = n1
    p2_b[...] = p2
    n2_b[...] = n2
    part1 = jnp.sum(p1, axis=0, keepdims=True) * (1.0 / N)
    part2 = jnp.sum(p2, axis=0, keepdims=True) * (1.0 / N)

    @pl.when(i == 0)
    def _():
        s1_b[...] = part1
        s2_b[...] = part2

    @pl.when(i != 0)
    def _():
        s1_b[...] += part1
        s2_b[...] += part2


def _d_call(raw, g1s, gn1s, g2s, gn2s, dv1, dv2, b1r, b2r):
    row_spec = pl.BlockSpec((RB, D), lambda i: (i, 0))
    dv_spec = pl.BlockSpec((RB, 1), lambda i: (i, 0))
    b_spec = pl.BlockSpec((1, D), lambda i: (0, 0))
    s_spec = pl.BlockSpec((1, D), lambda i: (0, 0))

    def raw_spec(b, c):
        return pl.BlockSpec((1, 1, RB, DH), lambda i, b=b, c=c: (b, c, i, 0))

    def g_spec(c):
        return pl.BlockSpec((1, RB, DH), lambda i, c=c: (c, i, 0))

    in_specs = ([raw_spec(b, c) for b in range(4) for c in range(2)]
                + [g_spec(c) for _ in range(4) for c in range(2)]
                + [dv_spec, dv_spec, b_spec, b_spec])
    return pl.pallas_call(
        _d_body,
        grid=(NB,),
        in_specs=in_specs,
        out_specs=[row_spec, row_spec, row_spec, row_spec, s_spec, s_spec],
        out_shape=[
            jax.ShapeDtypeStruct((N, D), jnp.float32),
            jax.ShapeDtypeStruct((N, D), jnp.float32),
            jax.ShapeDtypeStruct((N, D), jnp.float32),
            jax.ShapeDtypeStruct((N, D), jnp.float32),
            jax.ShapeDtypeStruct((1, D), jnp.float32),
            jax.ShapeDtypeStruct((1, D), jnp.float32),
        ],
    )(*([raw] * 8), g1s, g1s, gn1s, gn1s, g2s, g2s, gn2s, gn2s,
      dv1, dv2, b1r, b2r)


def kernel(x, edge_index1, edge_index2, neg_idx, W1, b1, W2, b2):
    s1x = jnp.stack([edge_index1[0], edge_index1[0] + N]).reshape(
        NC, NS, NSEG, SEG, CH)
    s2x = jnp.stack([edge_index2[0], edge_index2[0] + N]).reshape(
        NC, NS, NSEG, SEG, CH)
    d1x = edge_index1[1].reshape(NS, NSEG, SEG, CH)
    d2x = edge_index2[1].reshape(NS, NSEG, SEG, CH)
    e1d_cnt = edge_index1[1].reshape(NW, EPW // 16, 16)
    e2d_cnt = edge_index2[1].reshape(NW, EPW // 16, 16)
    ni_r = neg_idx.astype(jnp.int32)
    zeros = jnp.zeros((N, DH), jnp.float32)

    xn, dp1, dp2 = _a_kernel(x, ni_r, e1d_cnt, e2d_cnt)
    dp1t = dp1.reshape(NW, N).T
    dp2t = dp2.reshape(NW, N).T
    g1s, gn1s, g2s, gn2s, dv1, dv2 = _b_call(x, xn, W1, W2, dp1t, dp2t)
    raw = _c_kernel(
        g1s.reshape(NC * N, DH), gn1s.reshape(NC * N, DH),
        g2s.reshape(NC * N, DH), gn2s.reshape(NC * N, DH),
        zeros, s1x, d1x, s2x, d2x)
    p1, n1, p2, n2, s1, s2 = _d_call(
        raw, g1s, gn1s, g2s, gn2s, dv1, dv2,
        b1.reshape(1, D), b2.reshape(1, D))
    return (p1, n1, s1, p2, n2, s2)


# speedup vs baseline: 1.1203x; 1.1203x over previous
import functools

import jax
import jax.numpy as jnp
from jax import lax
from jax.experimental import pallas as pl
from jax.experimental.pallas import tpu as pltpu
from jax.experimental.pallas import tpu_sc as plsc

N = 10000
E = 320000
D = 128

NC = 2
NS = 16
NW = NC * NS
EPW = E // NW
CH = 50
NSEG = 5
SEG = 40
KB = 5
CHX = 80
NCH = EPW // CH
RPT = 624
TAIL0 = NS * RPT
TAIL = N - TAIL0
NXCH = (N + CHX - 1) // CHX

_mesh = plsc.VectorSubcoreMesh(core_axis_name="c", subcore_axis_name="s")


def _a_body(x_hbm, ni_hbm, e1d_hbm, e2d_hbm,
            xn_hbm, dp1_hbm, dp2_hbm,
            idx_v, rows_v, cnt_v, deg_v, sem):
    c = lax.axis_index("c")
    s = lax.axis_index("s")
    wid = s * NC + c

    for k in range((NXCH + NW - 1) // NW):
        jj = wid + NW * k

        @pl.when(jj < NXCH)
        def _():
            off = pl.multiple_of(jj * CHX, 8)
            pltpu.sync_copy(ni_hbm.at[pl.ds(off, CHX)], idx_v)
            pltpu.async_copy(x_hbm.at[idx_v], rows_v, sem).wait()
            pltpu.sync_copy(rows_v, xn_hbm.at[pl.ds(off, CHX)])

    ones16 = jnp.full((16,), 1.0, jnp.float32)
    zeros16 = jnp.zeros((16,), jnp.float32)
    for ed_hbm, dp_hbm in ((e1d_hbm, dp1_hbm), (e2d_hbm, dp2_hbm)):
        def zero_body(i, carry):
            deg_v[pl.ds(i * 16, 16)] = zeros16
            return carry
        lax.fori_loop(0, N // 16, zero_body, 0)
        pltpu.sync_copy(ed_hbm.at[wid], cnt_v)

        def cnt_body(i, carry):
            idx = cnt_v[i, :]
            plsc.addupdate_scatter(deg_v, [idx], ones16)
            return carry
        lax.fori_loop(0, EPW // 16, cnt_body, 0)
        pltpu.sync_copy(deg_v, dp_hbm.at[pl.ds(pl.multiple_of(wid * N, 8), N)])


_a_kernel = functools.partial(
    pl.kernel,
    out_type=(
        jax.ShapeDtypeStruct((N, D), jnp.float32),
        jax.ShapeDtypeStruct((NW * N,), jnp.float32),
        jax.ShapeDtypeStruct((NW * N,), jnp.float32),
    ),
    mesh=_mesh,
    scratch_types=[
        pltpu.VMEM((CHX,), jnp.int32),
        pltpu.VMEM((CHX, D), jnp.float32),
        pltpu.VMEM((EPW // 16, 16), jnp.int32),
        pltpu.VMEM((N,), jnp.float32),
        pltpu.SemaphoreType.DMA,
    ],
    compiler_params=pltpu.CompilerParams(needs_layout_passes=False),
)(_a_body)


RB = 400
NB = N // RB


def _b_body(x_b, xn_b, w1_b, w2_b, dp1_b, dp2_b,
            g1_b, gn1_b, g2_b, gn2_b, dv1_b, dv2_b):
    deg1 = jnp.sum(dp1_b[...], axis=1, keepdims=True) + 1.0
    deg2 = jnp.sum(dp2_b[...], axis=1, keepdims=True) + 1.0
    dv1 = lax.rsqrt(deg1)
    dv2 = lax.rsqrt(deg2)
    dv1_b[...] = dv1
    dv2_b[...] = dv2
    x = x_b[...]
    xn = xn_b[...]
    w1 = w1_b[...]
    w2 = w2_b[...]
    g1_b[...] = jnp.dot(x, w1, preferred_element_type=jnp.float32) * dv1
    gn1_b[...] = jnp.dot(xn, w1, preferred_element_type=jnp.float32) * dv1
    g2_b[...] = jnp.dot(x, w2, preferred_element_type=jnp.float32) * dv2
    gn2_b[...] = jnp.dot(xn, w2, preferred_element_type=jnp.float32) * dv2


def _b_call(x, xn, W1, W2, dp1, dp2):
    row_spec = pl.BlockSpec((RB, D), lambda i: (i, 0))
    w_spec = pl.BlockSpec((D, D), lambda i: (0, 0))
    dp_spec = pl.BlockSpec((RB, NW), lambda i: (i, 0))
    dv_spec = pl.BlockSpec((RB, 1), lambda i: (i, 0))
    return pl.pallas_call(
        _b_body,
        grid=(NB,),
        in_specs=[row_spec, row_spec, w_spec, w_spec, dp_spec, dp_spec],
        out_specs=[row_spec, row_spec, row_spec, row_spec, dv_spec, dv_spec],
        out_shape=[
            jax.ShapeDtypeStruct((N, D), jnp.float32),
            jax.ShapeDtypeStruct((N, D), jnp.float32),
            jax.ShapeDtypeStruct((N, D), jnp.float32),
            jax.ShapeDtypeStruct((N, D), jnp.float32),
            jax.ShapeDtypeStruct((N, 1), jnp.float32),
            jax.ShapeDtypeStruct((N, 1), jnp.float32),
        ],
    )(x, xn, W1, W2, dp1, dp2)


def _c_body(g1, gn1, g2, gn2, zeros_hbm, e1s, e1d, e2s, e2d,
            a00, a01, a10, a11, a20, a21, a30, a31,
            src_v, dst_v, rows, acc_sp, gsems, ssems):
    c = lax.axis_index("c")
    s = lax.axis_index("s")
    wid = s * NC + c
    row0 = pl.multiple_of(s * RPT, 8)
    outs = ((a00, a01), (a10, a11), (a20, a21), (a30, a31))
    edge_sets = (((e1s, e1d), ((0, g1), (1, gn1))),
                 ((e2s, e2d), ((2, g2), (3, gn2))))

    def _blk_copy(copy_fn):
        copy_fn(row0, RPT)

        @pl.when(s == 0)
        def _():
            copy_fn(TAIL0, TAIL)

    _blk_copy(lambda r, n: pltpu.sync_copy(
        zeros_hbm.at[pl.ds(r, n)], acc_sp.at[pl.ds(r, n)]))
    plsc.subcore_barrier()

    for (es, ed), branch_pair in edge_sets:
        for b, tbl in branch_pair:

            for seg in range(NSEG):
                pltpu.sync_copy(es.at[wid, seg], src_v)
                pltpu.sync_copy(ed.at[wid, seg], dst_v)
                for k in range(KB - 1):
                    pltpu.async_copy(tbl.at[src_v.at[k]], rows[k], gsems[k])

                def block(ib, carry):
                    jb = ib * KB
                    for k in range(KB):
                        j = jb + k
                        q = (k - 1) % KB
                        pltpu.make_async_copy(
                            tbl.at[src_v.at[j]], rows[k], gsems[k]).wait()
                        pltpu.async_copy(
                            rows[k], acc_sp.at[dst_v.at[j]], ssems[k],
                            add=True)

                        @pl.when(j > 0)
                        def _():
                            pltpu.make_async_copy(
                                rows[q], acc_sp.at[dst_v.at[j]], ssems[q]
                            ).wait()

                        @pl.when(j + KB - 1 < SEG)
                        def _():
                            pltpu.async_copy(
                                tbl.at[src_v.at[j + KB - 1]], rows[q],
                                gsems[q])
                    return carry
                lax.fori_loop(0, SEG // KB, block, 0)
                pltpu.make_async_copy(
                    rows[(SEG - 1) % KB], acc_sp.at[dst_v.at[SEG - 1]],
                    ssems[(SEG - 1) % KB]).wait()
            plsc.subcore_barrier()

            o0, o1 = outs[b]

            @pl.when(c == 0)
            def _():
                _blk_copy(lambda r, n: pltpu.sync_copy(
                    acc_sp.at[pl.ds(r, n)], o0.at[pl.ds(r, n)]))

            @pl.when(c != 0)
            def _():
                _blk_copy(lambda r, n: pltpu.sync_copy(
                    acc_sp.at[pl.ds(r, n)], o1.at[pl.ds(r, n)]))

            plsc.subcore_barrier()


_c_kernel = functools.partial(
    pl.kernel,
    out_type=tuple(jax.ShapeDtypeStruct((N, D), jnp.float32) for _ in range(8)),
    mesh=_mesh,
    scratch_types=[
        pltpu.VMEM((SEG, CH), jnp.int32),
        pltpu.VMEM((SEG, CH), jnp.int32),
        [pltpu.VMEM((CH, D), jnp.float32) for _ in range(KB)],
        pltpu.VMEM_SHARED((N, D), jnp.float32),
        [pltpu.SemaphoreType.DMA for _ in range(KB)],
        [pltpu.SemaphoreType.DMA for _ in range(KB)],
    ],
)(_c_body)


def _d_body(a00_b, a01_b, a10_b, a11_b, a20_b, a21_b, a30_b, a31_b,
            g1_b, gn1_b, g2_b, gn2_b,
            dv1_b, dv2_b, b1_b, b2_b,
            p1_b, n1_b, p2_b, n2_b, s1_b, s2_b):
    i = pl.program_id(0)
    dv1 = dv1_b[...]
    dv2 = dv2_b[...]
    b1 = b1_b[...]
    b2 = b2_b[...]
    r0 = a00_b[...] + a01_b[...]
    r1 = a10_b[...] + a11_b[...]
    r2 = a20_b[...] + a21_b[...]
    r3 = a30_b[...] + a31_b[...]
    p1 = jnp.maximum(dv1 * (r0 + g1_b[...]) + b1, 0.0)
    n1 = jnp.maximum(dv1 * (r1 - r0 + gn1_b[...]) + b1, 0.0)
    p2 = jnp.maximum(dv2 * (r2 - r1 + g2_b[...]) + b2, 0.0)
    n2 = jnp.maximum(dv2 * (r3 - r2 + gn2_b[...]) + b2, 0.0)
    p1_b[...] = p1
    n1_b[...] = n1
    p2_b[...] = p2
    n2_b[...] = n2
    part1 = jnp.sum(p1, axis=0, keepdims=True) * (1.0 / N)
    part2 = jnp.sum(p2, axis=0, keepdims=True) * (1.0 / N)

    @pl.when(i == 0)
    def _():
        s1_b[...] = part1
        s2_b[...] = part2

    @pl.when(i != 0)
    def _():
        s1_b[...] += part1
        s2_b[...] += part2


def _d_call(a00, a01, a10, a11, a20, a21, a30, a31,
            g1, gn1, g2, gn2, dv1, dv2, b1r, b2r):
    row_spec = pl.BlockSpec((RB, D), lambda i: (i, 0))
    dv_spec = pl.BlockSpec((RB, 1), lambda i: (i, 0))
    b_spec = pl.BlockSpec((1, D), lambda i: (0, 0))
    s_spec = pl.BlockSpec((1, D), lambda i: (0, 0))
    return pl.pallas_call(
        _d_body,
        grid=(NB,),
        in_specs=[row_spec] * 12 + [dv_spec, dv_spec, b_spec, b_spec],
        out_specs=[row_spec, row_spec, row_spec, row_spec, s_spec, s_spec],
        out_shape=[
            jax.ShapeDtypeStruct((N, D), jnp.float32),
            jax.ShapeDtypeStruct((N, D), jnp.float32),
            jax.ShapeDtypeStruct((N, D), jnp.float32),
            jax.ShapeDtypeStruct((N, D), jnp.float32),
            jax.ShapeDtypeStruct((1, D), jnp.float32),
            jax.ShapeDtypeStruct((1, D), jnp.float32),
        ],
    )(a00, a01, a10, a11, a20, a21, a30, a31,
      g1, gn1, g2, gn2, dv1, dv2, b1r, b2r)


def kernel(x, edge_index1, edge_index2, neg_idx, W1, b1, W2, b2):
    e1s = edge_index1[0].reshape(NW, NSEG, SEG, CH)
    e1d = edge_index1[1].reshape(NW, NSEG, SEG, CH)
    e2s = edge_index2[0].reshape(NW, NSEG, SEG, CH)
    e2d = edge_index2[1].reshape(NW, NSEG, SEG, CH)
    e1d_cnt = edge_index1[1].reshape(NW, EPW // 16, 16)
    e2d_cnt = edge_index2[1].reshape(NW, EPW // 16, 16)
    ni_r = neg_idx.astype(jnp.int32)
    zeros = jnp.zeros((N, D), jnp.float32)

    xn, dp1, dp2 = _a_kernel(x, ni_r, e1d_cnt, e2d_cnt)
    dp1t = dp1.reshape(NW, N).T
    dp2t = dp2.reshape(NW, N).T
    g1, gn1, g2, gn2, dv1, dv2 = _b_call(x, xn, W1, W2, dp1t, dp2t)
    a00, a01, a10, a11, a20, a21, a30, a31 = _c_kernel(
        g1, gn1, g2, gn2, zeros, e1s, e1d, e2s, e2d)
    p1, n1, p2, n2, s1, s2 = _d_call(
        a00, a01, a10, a11, a20, a21, a30, a31,
        g1, gn1, g2, gn2, dv1, dv2, b1.reshape(1, D), b2.reshape(1, D))
    return (p1, n1, s1, p2, n2, s2)
